# baseline (speedup 1.0000x reference)
import functools
import math

import jax
import jax.numpy as jnp
import numpy as np
from jax import lax
from jax.experimental import pallas as pl
from jax.experimental.pallas import tpu as pltpu
from jax.experimental.pallas import tpu_sc as plsc

D = 512
B = 4
S = 2048
NFLAT = B * S
SCALE = math.sqrt(D)

NC, NS, L = 2, 16, 16
NW = NC * NS
POS_PER_W = S // NW
CH = 32
NCHUNK = B * POS_PER_W // CH


def _positional_encoding() -> np.ndarray:
    position = np.arange(S, dtype=np.float32)[:, None]
    div_term = np.exp(
        np.arange(0, D, 2, dtype=np.float32) * (-math.log(10000.0) / D)
    )
    pe = np.zeros((S, D), dtype=np.float32)
    pe[:, 0::2] = np.sin(position * div_term)
    pe[:, 1::2] = np.cos(position * div_term)
    return pe


_PE_F32 = _positional_encoding()


def _make_kernel():
    mesh = plsc.VectorSubcoreMesh(core_axis_name="c", subcore_axis_name="s")

    @functools.partial(
        pl.kernel,
        mesh=mesh,
        out_type=jax.ShapeDtypeStruct((NFLAT, D), jnp.float32),
        scratch_types=[
            pltpu.VMEM((NCHUNK, CH), jnp.int32),
            pltpu.VMEM((CH, D), jnp.float32),
            pltpu.VMEM((CH, D), jnp.float32),
            pltpu.VMEM((CH, D), jnp.float32),
            pltpu.VMEM((CH, D), jnp.float32),
            pltpu.SemaphoreType.DMA,
            pltpu.SemaphoreType.DMA,
            pltpu.SemaphoreType.DMA,
            pltpu.SemaphoreType.DMA,
            pltpu.SemaphoreType.DMA,
            pltpu.SemaphoreType.DMA,
        ],
    )
    def emb(x_hbm, table_hbm, pe_hbm, out_hbm,
            idx_v, rows0, rows1, acc0, acc1,
            g0, g1, p0, p1, o0, o1):
        wid = lax.axis_index("s") * NC + lax.axis_index("c")
        pos0 = wid * POS_PER_W
        for b in range(B):
            pltpu.sync_copy(x_hbm.at[b, pl.ds(pos0, CH)], idx_v.at[2 * b])
            pltpu.sync_copy(
                x_hbm.at[b, pl.ds(pos0 + CH, CH)], idx_v.at[2 * b + 1])

        rows = (rows0, rows1)
        accs = (acc0, acc1)
        gsem = (g0, g1)
        psem = (p0, p1)
        osem = (o0, o1)
        g_h = [None, None]
        p_h = [None, None]
        o_h = [None, None]

        def pe_src(i):
            return pe_hbm.at[pl.ds(pos0 + (i % 2) * CH, CH)]

        def out_dst(i):
            return out_hbm.at[pl.ds((i // 2) * S + pos0 + (i % 2) * CH, CH)]

        g_h[0] = pltpu.async_copy(table_hbm.at[idx_v.at[0]], rows0, g0)
        p_h[0] = pltpu.async_copy(pe_src(0), acc0, p0)
        for i in range(NCHUNK):
            cur, nxt = i % 2, (i + 1) % 2
            if i + 1 < NCHUNK:
                if o_h[nxt] is not None:
                    o_h[nxt].wait()
                g_h[nxt] = pltpu.async_copy(
                    table_hbm.at[idx_v.at[i + 1]], rows[nxt], gsem[nxt])
                p_h[nxt] = pltpu.async_copy(pe_src(i + 1), accs[nxt],
                                            psem[nxt])
            g_h[cur].wait()
            p_h[cur].wait()

            def row(r, carry, cur=cur):
                for c in range(D // L):
                    sl = pl.ds(c * L, L)
                    plsc.addupdate(
                        accs[cur].at[r, sl], rows[cur][r, sl] * SCALE)
                return carry

            lax.fori_loop(0, CH, row, 0)
            o_h[cur] = pltpu.async_copy(accs[cur], out_dst(i), osem[cur])
        o_h[0].wait()
        o_h[1].wait()

    return emb


_emb = _make_kernel()


def kernel(x, table):
    pe = jnp.asarray(_PE_F32)
    out = _emb(x, table, pe)
    return out.reshape(B, S, D)


# speedup vs baseline: 1.4609x; 1.4609x over previous
import functools
import math

import jax
import jax.numpy as jnp
import numpy as np
from jax import lax
from jax.experimental import pallas as pl
from jax.experimental.pallas import tpu as pltpu
from jax.experimental.pallas import tpu_sc as plsc

D = 512
B = 4
S = 2048
NFLAT = B * S
SCALE = math.sqrt(D)

NC, NS, L = 2, 16, 16
NW = NC * NS
POS_PER_W = S // NW
CH = 32
NCHUNK = B * POS_PER_W // CH
NBUF = 3


def _positional_encoding() -> np.ndarray:
    position = np.arange(S, dtype=np.float32)[:, None]
    div_term = np.exp(
        np.arange(0, D, 2, dtype=np.float32) * (-math.log(10000.0) / D)
    )
    pe = np.zeros((S, D), dtype=np.float32)
    pe[:, 0::2] = np.sin(position * div_term)
    pe[:, 1::2] = np.cos(position * div_term)
    return pe


_PE_F32 = _positional_encoding()


def _make_kernel():
    mesh = plsc.VectorSubcoreMesh(core_axis_name="c", subcore_axis_name="s")

    @functools.partial(
        pl.kernel,
        mesh=mesh,
        out_type=jax.ShapeDtypeStruct((NFLAT, D), jnp.float32),
        scratch_types=[
            pltpu.VMEM((NCHUNK, CH), jnp.int32),
            pltpu.VMEM((POS_PER_W, D), jnp.float32),
            pltpu.VMEM((CH, D), jnp.float32),
            pltpu.VMEM((CH, D), jnp.float32),
            pltpu.VMEM((CH, D), jnp.float32),
            pltpu.SemaphoreType.DMA,
            pltpu.SemaphoreType.DMA,
            pltpu.SemaphoreType.DMA,
            pltpu.SemaphoreType.DMA,
            pltpu.SemaphoreType.DMA,
            pltpu.SemaphoreType.DMA,
            pltpu.SemaphoreType.DMA,
            pltpu.SemaphoreType.DMA,
        ],
    )
    def emb(x_hbm, table_hbm, pe_hbm, out_hbm,
            idx_v, pe_v, rows0, rows1, rows2,
            isem, psem, g0, g1, g2, o0, o1, o2):
        wid = lax.axis_index("s") * NC + lax.axis_index("c")
        pos0 = wid * POS_PER_W

        i_h = []
        for b in range(B):
            i_h.append(pltpu.async_copy(
                x_hbm.at[b, pl.ds(pos0, CH)], idx_v.at[2 * b], isem))
            i_h.append(pltpu.async_copy(
                x_hbm.at[b, pl.ds(pos0 + CH, CH)], idx_v.at[2 * b + 1], isem))
        for h in i_h:
            h.wait()

        rows = (rows0, rows1, rows2)
        gsem = (g0, g1, g2)
        osem = (o0, o1, o2)
        g_h = [None, None, None]
        o_h = [None, None, None]

        def out_dst(i):
            return out_hbm.at[pl.ds((i // 2) * S + pos0 + (i % 2) * CH, CH)]

        g_h[0] = pltpu.async_copy(table_hbm.at[idx_v.at[0]], rows[0], g0)
        p_h = pltpu.async_copy(pe_hbm.at[pl.ds(pos0, POS_PER_W)], pe_v, psem)
        g_h[1] = pltpu.async_copy(table_hbm.at[idx_v.at[1]], rows[1], g1)

        for i in range(NCHUNK):
            cur = i % NBUF
            g_h[cur].wait()
            if i == 0:
                p_h.wait()
            if i + 2 < NCHUNK:
                n2 = (i + 2) % NBUF
                if o_h[n2] is not None:
                    o_h[n2].wait()
                g_h[n2] = pltpu.async_copy(
                    table_hbm.at[idx_v.at[i + 2]], rows[n2], gsem[n2])

            pe_base = (i % 2) * CH

            def row(r, carry, cur=cur, pe_base=pe_base):
                for c in range(D // L):
                    sl = pl.ds(c * L, L)
                    rows[cur][r, sl] = (
                        rows[cur][r, sl] * SCALE + pe_v[pe_base + r, sl])
                return carry

            lax.fori_loop(0, CH, row, 0)
            o_h[cur] = pltpu.async_copy(rows[cur], out_dst(i), osem[cur])
        for h in o_h:
            h.wait()

    return emb


_emb = _make_kernel()


def kernel(x, table):
    pe = jnp.asarray(_PE_F32)
    out = _emb(x, table, pe)
    return out.reshape(B, S, D)


# speedup vs baseline: 1.6138x; 1.1047x over previous
import functools
import math

import jax
import jax.numpy as jnp
import numpy as np
from jax import lax
from jax.experimental import pallas as pl
from jax.experimental.pallas import tpu as pltpu
from jax.experimental.pallas import tpu_sc as plsc

D = 512
B = 4
S = 2048
NFLAT = B * S
SCALE = math.sqrt(D)

NC, NS, L = 2, 16, 16
NW = NC * NS
POS_PER_W = S // NW


def _positional_encoding() -> np.ndarray:
    position = np.arange(S, dtype=np.float32)[:, None]
    div_term = np.exp(
        np.arange(0, D, 2, dtype=np.float32) * (-math.log(10000.0) / D)
    )
    pe = np.zeros((S, D), dtype=np.float32)
    pe[:, 0::2] = np.sin(position * div_term)
    pe[:, 1::2] = np.cos(position * div_term)
    return pe


_PE_F32 = _positional_encoding()


def _make_kernel():
    mesh = plsc.VectorSubcoreMesh(core_axis_name="c", subcore_axis_name="s")

    @functools.partial(
        pl.kernel,
        mesh=mesh,
        out_type=jax.ShapeDtypeStruct((NFLAT, D), jnp.float32),
        scratch_types=[
            pltpu.VMEM((B, POS_PER_W), jnp.int32),
            pltpu.VMEM((POS_PER_W, D), jnp.float32),
            pltpu.VMEM((POS_PER_W, D), jnp.float32),
            pltpu.VMEM((POS_PER_W, D), jnp.float32),
            pltpu.SemaphoreType.DMA,
            pltpu.SemaphoreType.DMA,
            pltpu.SemaphoreType.DMA,
            pltpu.SemaphoreType.DMA,
            pltpu.SemaphoreType.DMA,
            pltpu.SemaphoreType.DMA,
        ],
    )
    def emb(x_hbm, table_hbm, pe_hbm, out_hbm,
            idx_v, pe_v, rows0, rows1, isem, psem, g0, g1, o0, o1):
        wid = lax.axis_index("s") * NC + lax.axis_index("c")
        pos0 = wid * POS_PER_W

        i_h = [
            pltpu.async_copy(
                x_hbm.at[b, pl.ds(pos0, POS_PER_W)], idx_v.at[b], isem)
            for b in range(B)
        ]
        for h in i_h:
            h.wait()

        rows = (rows0, rows1)
        gsem = (g0, g1)
        osem = (o0, o1)
        g_h = [None, None]
        o_h = [None, None]
        g_h[0] = pltpu.async_copy(table_hbm.at[idx_v.at[0]], rows0, g0)
        p_h = pltpu.async_copy(pe_hbm.at[pl.ds(pos0, POS_PER_W)], pe_v, psem)
        for b in range(B):
            cur, nxt = b % 2, (b + 1) % 2
            if b + 1 < B:
                if o_h[nxt] is not None:
                    o_h[nxt].wait()
                g_h[nxt] = pltpu.async_copy(
                    table_hbm.at[idx_v.at[b + 1]], rows[nxt], gsem[nxt])
            g_h[cur].wait()
            if b == 0:
                p_h.wait()

            def row(r, carry, cur=cur):
                for c in range(D // L):
                    sl = pl.ds(c * L, L)
                    rows[cur][r, sl] = rows[cur][r, sl] * SCALE + pe_v[r, sl]
                return carry

            lax.fori_loop(0, POS_PER_W, row, 0)
            o_h[cur] = pltpu.async_copy(
                rows[cur], out_hbm.at[pl.ds(b * S + pos0, POS_PER_W)],
                osem[cur])
        o_h[0].wait()
        o_h[1].wait()

    return emb


_emb = _make_kernel()


def kernel(x, table):
    pe = jnp.asarray(_PE_F32)
    out = _emb(x, table, pe)
    return out.reshape(B, S, D)
